# Initial kernel scaffold; baseline (speedup 1.0000x reference)
import jax
import jax.numpy as jnp
from jax.experimental import pallas as pl


def kernel(node_feat, edge_feat, edge_index, edge_type, W_node, b_node, W_edge, b_edge, W_attn, b_attn, W_self):
    raise NotImplementedError("write your pallas kernel here")


# speedup vs baseline: 1.2142x; 1.2142x over previous
import functools

import jax
import jax.numpy as jnp
from jax.experimental import pallas as pl

H = 4
HD = 32
F = H * HD


def _proj_body(x_ref, wn_ref, b_ref, a_ref, s_ref, node_ref, ascal_ref, sl_ref):
    x = x_ref[...]
    node = jnp.dot(x, wn_ref[...], preferred_element_type=jnp.float32) + b_ref[...]
    node_ref[...] = node
    ascal_ref[...] = jnp.dot(node, a_ref[...], preferred_element_type=jnp.float32)
    sl_ref[...] = jnp.dot(node, s_ref[...], preferred_element_type=jnp.float32)


def _edge_body(ef_ref, we_ref, be_ref, ae_ref, edge_ref, aedge_ref):
    ef = ef_ref[...]
    edge = jnp.einsum("rd,rdc->rc", ef, we_ref[...],
                      preferred_element_type=jnp.float32) + be_ref[...]
    edge_ref[...] = edge
    aedge_ref[...] = jnp.dot(edge, ae_ref[...], preferred_element_type=jnp.float32)


def _final_body(acc_ref, den_ref, sl_ref, out_ref):
    den = den_ref[...]
    safe = jnp.where(den > 0, den, 1.0)
    inv = (1.0 / safe)[:, :, None]
    agg = acc_ref[...].reshape(-1, H, HD) * inv
    v = agg.reshape(-1, F) + sl_ref[...]
    out_ref[...] = jnp.where(v > 0, v, 0.01 * v)


def kernel(node_feat, edge_feat, edge_index, edge_type, W_node, b_node,
           W_edge, b_edge, W_attn, b_attn, W_self):
    n = node_feat.shape[0]
    nd = node_feat.shape[1]
    r = edge_feat.shape[0]
    ed = edge_feat.shape[1]

    wn_flat = W_node.transpose(2, 0, 1).reshape(nd, F)
    b_flat = b_node.reshape(F)
    wa = W_attn[:, 0, :]
    eye_h = jnp.eye(H, dtype=jnp.float32)
    a_src = wa[:, :HD]
    a_dst = wa[:, 2 * HD:]
    A = jnp.concatenate([
        (a_src[:, :, None] * eye_h[:, None, :]).reshape(F, H),
        (a_dst[:, :, None] * eye_h[:, None, :]).reshape(F, H),
    ], axis=1)
    S = (W_self.transpose(0, 2, 1)[:, None, :, :] *
         eye_h[:, :, None, None]).transpose(0, 2, 1, 3).reshape(F, F)
    we = W_edge.transpose(1, 3, 0, 2).reshape(r, ed, F)
    be_flat = b_edge.transpose(1, 0, 2).reshape(r, F)
    a_edge_w = (wa[:, HD:2 * HD][:, :, None] * eye_h[:, None, :]).reshape(F, H)

    BN = 2000
    grid = (n // BN,)
    node_flat, ascal, sl = pl.pallas_call(
        _proj_body,
        grid=grid,
        in_specs=[
            pl.BlockSpec((BN, nd), lambda i: (i, 0)),
            pl.BlockSpec((nd, F), lambda i: (0, 0)),
            pl.BlockSpec((F,), lambda i: (0,)),
            pl.BlockSpec((F, 2 * H), lambda i: (0, 0)),
            pl.BlockSpec((F, F), lambda i: (0, 0)),
        ],
        out_specs=[
            pl.BlockSpec((BN, F), lambda i: (i, 0)),
            pl.BlockSpec((BN, 2 * H), lambda i: (i, 0)),
            pl.BlockSpec((BN, F), lambda i: (i, 0)),
        ],
        out_shape=[
            jax.ShapeDtypeStruct((n, F), jnp.float32),
            jax.ShapeDtypeStruct((n, 2 * H), jnp.float32),
            jax.ShapeDtypeStruct((n, F), jnp.float32),
        ],
    )(node_feat, wn_flat, b_flat, A, S)

    edge_flat, aedge = pl.pallas_call(
        _edge_body,
        out_shape=[
            jax.ShapeDtypeStruct((r, F), jnp.float32),
            jax.ShapeDtypeStruct((r, H), jnp.float32),
        ],
    )(edge_feat, we, be_flat, a_edge_w)

    src = edge_index[0]
    dst = edge_index[1]
    att = (ascal[src, :H] + aedge[edge_type] + ascal[dst, H:]
           + b_attn[:, 0][None, :])
    att = jnp.where(att > 0, att, 0.01 * att)
    p = jnp.exp(att)
    denom = jax.ops.segment_sum(p, dst, num_segments=n)
    w = jnp.repeat(p, HD, axis=1) * edge_flat[edge_type]
    acc = jax.ops.segment_sum(node_flat[src] * w, dst, num_segments=n)

    out = pl.pallas_call(
        _final_body,
        grid=grid,
        in_specs=[
            pl.BlockSpec((BN, F), lambda i: (i, 0)),
            pl.BlockSpec((BN, H), lambda i: (i, 0)),
            pl.BlockSpec((BN, F), lambda i: (i, 0)),
        ],
        out_specs=pl.BlockSpec((BN, F), lambda i: (i, 0)),
        out_shape=jax.ShapeDtypeStruct((n, F), jnp.float32),
    )(acc, denom, sl)
    return out


# speedup vs baseline: 29.9843x; 24.6941x over previous
import functools

import jax
import jax.numpy as jnp
from jax import lax
from jax.experimental import pallas as pl
from jax.experimental.pallas import tpu as pltpu
from jax.experimental.pallas import tpu_sc as plsc

H = 4
HD = 32
F = H * HD
NC = 2
NS = 16
L = 16
NW = NC * NS
SB = 2000
B = 80


def _proj_body(x_ref, wn_ref, b_ref, a_ref, s_ref, node_ref, ascal_ref, sl_ref):
    x = x_ref[...]
    node = jnp.dot(x, wn_ref[...], preferred_element_type=jnp.float32) + b_ref[...]
    node_ref[...] = node
    ascal_ref[...] = jnp.dot(node, a_ref[...], preferred_element_type=jnp.float32)
    sl_ref[...] = jnp.dot(node, s_ref[...], preferred_element_type=jnp.float32)


def _edge_body(ef_ref, we_ref, be_ref, ae_ref, ba_ref, edge_ref, aedge_ref):
    ef = ef_ref[...]
    edge = jnp.einsum("rd,rdc->rc", ef, we_ref[...],
                      preferred_element_type=jnp.float32) + be_ref[...]
    edge_ref[...] = edge
    aedge_ref[...] = (jnp.dot(edge, ae_ref[...], preferred_element_type=jnp.float32)
                      + ba_ref[...])


def _final_body(acc_ref, den_ref, m4_ref, sl_ref, out_ref):
    acc = jnp.sum(acc_ref[...], axis=0)
    den = jnp.dot(den_ref[...], m4_ref[...],
                  preferred_element_type=jnp.float32)
    inv = (1.0 / jnp.where(den > 0, den, 1.0))[:, :, None]
    v = (acc.reshape(-1, H, HD) * inv).reshape(-1, F) + sl_ref[...]
    out_ref[...] = jnp.where(v > 0, v, 0.01 * v)


def _att_p(ascal_t, aedge_t, sv, dv, tv):
    sv8 = sv * 8
    dv8 = dv * 8
    tv4 = tv * 4
    out = []
    for h in range(H):
        a = (plsc.load_gather(ascal_t, [sv8 + h])
             + plsc.load_gather(ascal_t, [dv8 + (H + h)])
             + plsc.load_gather(aedge_t, [tv4 + h]))
        a = jnp.maximum(a, a * 0.01)
        out.append(jnp.exp(a))
    return out


def _make_sc_den(n, e):
    epw = e // NW
    nsb = epw // SB

    def body(src_hbm, dst_hbm, typ_hbm, ascal_hbm, aedge_hbm, zden_hbm,
             den_out, p_out, src_f, dst_f, typ_f, p_v, ascal_t, aedge_t, den_t):
        c = lax.axis_index("c")
        s = lax.axis_index("s")
        wid = c * NS + s
        pltpu.sync_copy(ascal_hbm, ascal_t)
        pltpu.sync_copy(aedge_hbm, aedge_t)
        pltpu.sync_copy(zden_hbm, den_t)

        iota = lax.iota(jnp.int32, L)
        lane_h = jnp.bitwise_and(iota, 3)
        lane_m = iota < H

        def super_body(sb, carry):
            base = wid * epw + sb * SB
            pltpu.sync_copy(src_hbm.at[pl.ds(base, SB)], src_f)
            pltpu.sync_copy(dst_hbm.at[pl.ds(base, SB)], dst_f)
            pltpu.sync_copy(typ_hbm.at[pl.ds(base, SB)], typ_f)

            def batch_body(b, carry2):
                def att_g(g, carry3):
                    le = b * B + g * L
                    rows = g * L + iota
                    sv = plsc.load_gather(src_f, [le + iota])
                    dv = plsc.load_gather(dst_f, [le + iota])
                    tv = plsc.load_gather(typ_f, [le + iota])
                    ps = _att_p(ascal_t, aedge_t, sv, dv, tv)
                    for h in range(H):
                        plsc.store_scatter(p_v, [rows * 4 + h], ps[h])
                    return carry3
                lax.fori_loop(0, B // L, att_g, 0)

                def den_e(ei, carry3):
                    ge = jnp.full((L,), b * B + ei, jnp.int32)
                    dvs = plsc.load_gather(dst_f, [ge])
                    p4 = plsc.load_gather(p_v, [jnp.full((L,), ei * 4, jnp.int32)
                                                + lane_h])
                    plsc.addupdate_scatter(den_t, [dvs * 4 + lane_h], p4,
                                           mask=lane_m)
                    return carry3
                lax.fori_loop(0, B, den_e, 0)
                pltpu.sync_copy(
                    p_v, p_out.at[pl.ds((base + b * B) * H, B * H)])
                return carry2
            lax.fori_loop(0, SB // B, batch_body, 0)
            return carry
        lax.fori_loop(0, nsb, super_body, 0)

        pltpu.sync_copy(den_t, den_out.at[pl.ds(wid * n * H, n * H)])

    return pl.kernel(
        body,
        out_type=[
            jax.ShapeDtypeStruct((NW * n * H,), jnp.float32),
            jax.ShapeDtypeStruct((e * H,), jnp.float32),
        ],
        compiler_params=pltpu.CompilerParams(needs_layout_passes=False),
        mesh=plsc.VectorSubcoreMesh(core_axis_name="c", subcore_axis_name="s"),
        scratch_types=[
            pltpu.VMEM((SB,), jnp.int32),
            pltpu.VMEM((SB,), jnp.int32),
            pltpu.VMEM((SB,), jnp.int32),
            pltpu.VMEM((B * H,), jnp.float32),
            pltpu.VMEM((n * 8,), jnp.float32),
            pltpu.VMEM((16 * H,), jnp.float32),
            pltpu.VMEM((n * H,), jnp.float32),
        ],
    )


def _make_sc_msg(n, e, r):
    epw = e // NW
    nsb = epw // SB
    npc = n // NS

    def body(src_hbm, dst_hbm, typ_hbm, p_hbm, node_hbm,
             edge_hbm, z128_hbm, acc_out,
             src_f, dst_f, typ_f, p_f, node_v, edge_t, acc_sh):
        c = lax.axis_index("c")
        s = lax.axis_index("s")
        wid = c * NS + s
        pltpu.sync_copy(edge_hbm, edge_t)
        pltpu.sync_copy(z128_hbm, acc_sh.at[pl.ds(s * npc, npc)])
        plsc.subcore_barrier()

        iota = lax.iota(jnp.int32, L)

        def super_body(sb, carry):
            base = wid * epw + sb * SB
            pltpu.sync_copy(src_hbm.at[pl.ds(base, SB)], src_f)
            pltpu.sync_copy(dst_hbm.at[pl.ds(base, SB)], dst_f)
            pltpu.sync_copy(typ_hbm.at[pl.ds(base, SB)], typ_f)
            pltpu.sync_copy(p_hbm.at[pl.ds(base * H, SB * H)], p_f)

            def batch_body(b, carry2):
                def gat_g(g, carry3):
                    sv = plsc.load_gather(src_f, [b * B + g * L + iota])
                    pltpu.sync_copy(node_hbm.at[sv],
                                    node_v.at[pl.ds(g * L, L)])
                    return carry3
                lax.fori_loop(0, B // L, gat_g, 0)

                def msg_e(ei, carry3):
                    fe = jnp.full((L,), ei, jnp.int32)
                    ge = jnp.full((L,), b * B + ei, jnp.int32)
                    tv = plsc.load_gather(typ_f, [ge])
                    ps = [plsc.load_gather(p_f, [jnp.full((L,), (b * B + ei) * H + h,
                                                          jnp.int32)])
                          for h in range(H)]
                    for j in range(F // L):
                        cols = j * L + iota
                        nv = plsc.load_gather(node_v, [fe, cols])
                        ev = plsc.load_gather(edge_t, [tv, cols])
                        plsc.store_scatter(node_v, [fe, cols],
                                           nv * ev * ps[j // (HD // L)])
                    return carry3
                lax.fori_loop(0, B, msg_e, 0)

                def sca_g(g, carry3):
                    dv = plsc.load_gather(dst_f, [b * B + g * L + iota])
                    pltpu.sync_copy(node_v.at[pl.ds(g * L, L)],
                                    acc_sh.at[dv], add=True)
                    return carry3
                lax.fori_loop(0, B // L, sca_g, 0)
                return carry2
            lax.fori_loop(0, SB // B, batch_body, 0)
            return carry
        lax.fori_loop(0, nsb, super_body, 0)

        plsc.subcore_barrier()

        @pl.when(s == 0)
        def _():
            pltpu.sync_copy(acc_sh, acc_out.at[c])

    return pl.kernel(
        body,
        out_type=jax.ShapeDtypeStruct((NC, n, F), jnp.float32),
        compiler_params=pltpu.CompilerParams(needs_layout_passes=False),
        mesh=plsc.VectorSubcoreMesh(core_axis_name="c", subcore_axis_name="s"),
        scratch_types=[
            pltpu.VMEM((SB,), jnp.int32),
            pltpu.VMEM((SB,), jnp.int32),
            pltpu.VMEM((SB,), jnp.int32),
            pltpu.VMEM((SB * H,), jnp.float32),
            pltpu.VMEM((B, F), jnp.float32),
            pltpu.VMEM((r, F), jnp.float32),
            pltpu.VMEM_SHARED((n, F), jnp.float32),
        ],
    )


def kernel(node_feat, edge_feat, edge_index, edge_type, W_node, b_node,
           W_edge, b_edge, W_attn, b_attn, W_self):
    n = node_feat.shape[0]
    nd = node_feat.shape[1]
    r = edge_feat.shape[0]
    ed = edge_feat.shape[1]
    e = edge_type.shape[0]

    wn_flat = W_node.transpose(2, 0, 1).reshape(nd, F)
    b_flat = b_node.reshape(F)
    wa = W_attn[:, 0, :]
    eye_h = jnp.eye(H, dtype=jnp.float32)
    A = jnp.concatenate([
        (wa[:, :HD][:, :, None] * eye_h[:, None, :]).reshape(F, H),
        (wa[:, 2 * HD:][:, :, None] * eye_h[:, None, :]).reshape(F, H),
    ], axis=1)
    S = (W_self.transpose(0, 2, 1)[:, None, :, :] *
         eye_h[:, :, None, None]).transpose(0, 2, 1, 3).reshape(F, F)
    we = W_edge.transpose(1, 3, 0, 2).reshape(r, ed, F)
    be_flat = b_edge.transpose(1, 0, 2).reshape(r, F)
    a_edge_w = (wa[:, HD:2 * HD][:, :, None] * eye_h[:, None, :]).reshape(F, H)
    ba_row = b_attn[:, 0][None, :]

    BN = 2000
    grid = (n // BN,)
    node_flat, ascal, sl = pl.pallas_call(
        _proj_body,
        grid=grid,
        in_specs=[
            pl.BlockSpec((BN, nd), lambda i: (i, 0)),
            pl.BlockSpec((nd, F), lambda i: (0, 0)),
            pl.BlockSpec((F,), lambda i: (0,)),
            pl.BlockSpec((F, 2 * H), lambda i: (0, 0)),
            pl.BlockSpec((F, F), lambda i: (0, 0)),
        ],
        out_specs=[
            pl.BlockSpec((BN, F), lambda i: (i, 0)),
            pl.BlockSpec((BN, 2 * H), lambda i: (i, 0)),
            pl.BlockSpec((BN, F), lambda i: (i, 0)),
        ],
        out_shape=[
            jax.ShapeDtypeStruct((n, F), jnp.float32),
            jax.ShapeDtypeStruct((n, 2 * H), jnp.float32),
            jax.ShapeDtypeStruct((n, F), jnp.float32),
        ],
    )(node_feat, wn_flat, b_flat, A, S)

    edge_flat, aedge = pl.pallas_call(
        _edge_body,
        out_shape=[
            jax.ShapeDtypeStruct((r, F), jnp.float32),
            jax.ShapeDtypeStruct((r, H), jnp.float32),
        ],
    )(edge_feat, we, be_flat, a_edge_w, ba_row)

    src = edge_index[0]
    dst = edge_index[1]
    ascal_f = ascal.reshape(n * 8)
    aedge_f = aedge.reshape(r * H)
    zden = jnp.zeros((n * H,), jnp.float32)
    z128 = jnp.zeros((n // NS, F), jnp.float32)

    den_parts, p_edges = _make_sc_den(n, e)(src, dst, edge_type, ascal_f,
                                            aedge_f, zden)
    acc = _make_sc_msg(n, e, r)(src, dst, edge_type, p_edges, node_flat,
                                edge_flat, z128)
    den2d = den_parts.reshape(NW, n, H).transpose(1, 0, 2).reshape(n, NW * H)
    m4 = jnp.tile(eye_h, (NW, 1))

    out = pl.pallas_call(
        _final_body,
        grid=grid,
        in_specs=[
            pl.BlockSpec((NC, BN, F), lambda i: (0, i, 0)),
            pl.BlockSpec((BN, NW * H), lambda i: (i, 0)),
            pl.BlockSpec((NW * H, H), lambda i: (0, 0)),
            pl.BlockSpec((BN, F), lambda i: (i, 0)),
        ],
        out_specs=pl.BlockSpec((BN, F), lambda i: (i, 0)),
        out_shape=jax.ShapeDtypeStruct((n, F), jnp.float32),
    )(acc, den2d, m4, sl)
    return out


# speedup vs baseline: 38.9434x; 1.2988x over previous
import functools

import jax
import jax.numpy as jnp
from jax import lax
from jax.experimental import pallas as pl
from jax.experimental.pallas import tpu as pltpu
from jax.experimental.pallas import tpu_sc as plsc

H = 4
HD = 32
F = H * HD
NC = 2
NS = 16
L = 16
NW = NC * NS
SB = 2000
B = 80


def _proj_body(x_ref, wn_ref, b_ref, a_ref, s_ref, node_ref, ascal_ref, sl_ref):
    x = x_ref[...]
    node = jnp.dot(x, wn_ref[...], preferred_element_type=jnp.float32) + b_ref[...]
    node_ref[...] = node
    ascal_ref[...] = jnp.dot(node, a_ref[...], preferred_element_type=jnp.float32)
    sl_ref[...] = jnp.dot(node, s_ref[...], preferred_element_type=jnp.float32)


def _edge_body(ef_ref, we_ref, be_ref, ae_ref, ba_ref, edge_ref, aedge_ref):
    ef = ef_ref[...]
    edge = jnp.einsum("rd,rdc->rc", ef, we_ref[...],
                      preferred_element_type=jnp.float32) + be_ref[...]
    edge_ref[...] = edge
    aedge_ref[...] = (jnp.dot(edge, ae_ref[...], preferred_element_type=jnp.float32)
                      + ba_ref[...])


def _final_body(acc_ref, den_ref, m4_ref, sl_ref, out_ref):
    acc = jnp.sum(acc_ref[...], axis=0)
    den = jnp.dot(den_ref[...], m4_ref[...],
                  preferred_element_type=jnp.float32)
    inv = (1.0 / jnp.where(den > 0, den, 1.0))[:, :, None]
    v = (acc.reshape(-1, H, HD) * inv).reshape(-1, F) + sl_ref[...]
    out_ref[...] = jnp.where(v > 0, v, 0.01 * v)


def _att_p(ascal_t, aedge_t, sv, dv, tv):
    sv8 = sv * 8
    dv8 = dv * 8
    tv4 = tv * 4
    out = []
    for h in range(H):
        a = (plsc.load_gather(ascal_t, [sv8 + h])
             + plsc.load_gather(ascal_t, [dv8 + (H + h)])
             + plsc.load_gather(aedge_t, [tv4 + h]))
        a = jnp.maximum(a, a * 0.01)
        out.append(jnp.exp(a))
    return out


def _make_sc_den(n, e):
    epw = e // NW
    nsb = epw // SB

    def body(src_hbm, dst_hbm, typ_hbm, ascal_hbm, aedge_hbm, zden_hbm,
             den_out, p_out, src_f, dst_f, typ_f, p_v, ascal_t, aedge_t, den_t):
        c = lax.axis_index("c")
        s = lax.axis_index("s")
        wid = c * NS + s
        pltpu.sync_copy(ascal_hbm, ascal_t)
        pltpu.sync_copy(aedge_hbm, aedge_t)
        pltpu.sync_copy(zden_hbm, den_t)

        iota = lax.iota(jnp.int32, L)
        lane_h = jnp.bitwise_and(iota, 3)
        lane_m = iota < H

        def super_body(sb, carry):
            base = wid * epw + sb * SB
            pltpu.sync_copy(src_hbm.at[pl.ds(base, SB)], src_f)
            pltpu.sync_copy(dst_hbm.at[pl.ds(base, SB)], dst_f)
            pltpu.sync_copy(typ_hbm.at[pl.ds(base, SB)], typ_f)

            def batch_body(b, carry2):
                def att_g(g, carry3):
                    le = b * B + g * L
                    rows = g * L + iota
                    sv = plsc.load_gather(src_f, [le + iota])
                    dv = plsc.load_gather(dst_f, [le + iota])
                    tv = plsc.load_gather(typ_f, [le + iota])
                    ps = _att_p(ascal_t, aedge_t, sv, dv, tv)
                    for h in range(H):
                        plsc.store_scatter(p_v, [rows * 4 + h], ps[h])
                    return carry3
                lax.fori_loop(0, B // L, att_g, 0)

                def den_e(ei, carry3):
                    ge = jnp.full((L,), b * B + ei, jnp.int32)
                    dvs = plsc.load_gather(dst_f, [ge])
                    p4 = plsc.load_gather(p_v, [jnp.full((L,), ei * 4, jnp.int32)
                                                + lane_h])
                    plsc.addupdate_scatter(den_t, [dvs * 4 + lane_h], p4,
                                           mask=lane_m)
                    return carry3
                lax.fori_loop(0, B, den_e, 0)
                pltpu.sync_copy(
                    p_v, p_out.at[pl.ds((base + b * B) * H, B * H)])
                return carry2
            lax.fori_loop(0, SB // B, batch_body, 0)
            return carry
        lax.fori_loop(0, nsb, super_body, 0)

        pltpu.sync_copy(den_t, den_out.at[pl.ds(wid * n * H, n * H)])

    return pl.kernel(
        body,
        out_type=[
            jax.ShapeDtypeStruct((NW * n * H,), jnp.float32),
            jax.ShapeDtypeStruct((e * H,), jnp.float32),
        ],
        compiler_params=pltpu.CompilerParams(needs_layout_passes=False),
        mesh=plsc.VectorSubcoreMesh(core_axis_name="c", subcore_axis_name="s"),
        scratch_types=[
            pltpu.VMEM((SB,), jnp.int32),
            pltpu.VMEM((SB,), jnp.int32),
            pltpu.VMEM((SB,), jnp.int32),
            pltpu.VMEM((B * H,), jnp.float32),
            pltpu.VMEM((n * 8,), jnp.float32),
            pltpu.VMEM((16 * H,), jnp.float32),
            pltpu.VMEM((n * H,), jnp.float32),
        ],
    )


def _make_sc_msg(n, e, r):
    epw = e // NW
    nsb = epw // SB
    npc = n // NS

    def body(src_hbm, dst_hbm, typ_hbm, p_hbm, node_hbm,
             edge_hbm, z128_hbm, acc_out,
             src_f, dst_f, typ_f, p_f, node_v, dst_b, edge_t, acc_sh):
        c = lax.axis_index("c")
        s = lax.axis_index("s")
        wid = c * NS + s
        pltpu.sync_copy(edge_hbm, edge_t)
        pltpu.sync_copy(z128_hbm, acc_sh.at[pl.ds(s * npc, npc)])
        plsc.subcore_barrier()

        iota = lax.iota(jnp.int32, L)

        def super_body(sb, carry):
            base = wid * epw + sb * SB
            pltpu.sync_copy(src_hbm.at[pl.ds(base, SB)], src_f)
            pltpu.sync_copy(dst_hbm.at[pl.ds(base, SB)], dst_f)
            pltpu.sync_copy(typ_hbm.at[pl.ds(base, SB)], typ_f)
            pltpu.sync_copy(p_hbm.at[pl.ds(base * H, SB * H)], p_f)

            def batch_body(b, carry2):
                pltpu.sync_copy(node_hbm.at[src_f.at[pl.ds(b * B, B)]], node_v)
                for g in range(B // L):
                    dst_b[pl.ds(g * L, L)] = plsc.load_gather(
                        dst_f, [b * B + g * L + iota])

                def msg_e(ei, carry3):
                    fe = jnp.full((L,), ei, jnp.int32)
                    ge = jnp.full((L,), b * B + ei, jnp.int32)
                    tv = plsc.load_gather(typ_f, [ge])
                    ps = [plsc.load_gather(p_f, [jnp.full((L,), (b * B + ei) * H + h,
                                                          jnp.int32)])
                          for h in range(H)]
                    for j in range(F // L):
                        cols = j * L + iota
                        nv = plsc.load_gather(node_v, [fe, cols])
                        ev = plsc.load_gather(edge_t, [tv, cols])
                        plsc.store_scatter(node_v, [fe, cols],
                                           nv * ev * ps[j // (HD // L)])
                    return carry3
                lax.fori_loop(0, B, msg_e, 0)

                pltpu.sync_copy(node_v, acc_sh.at[dst_b], add=True)
                return carry2
            lax.fori_loop(0, SB // B, batch_body, 0)
            return carry
        lax.fori_loop(0, nsb, super_body, 0)

        plsc.subcore_barrier()

        @pl.when(s == 0)
        def _():
            pltpu.sync_copy(acc_sh, acc_out.at[c])

    return pl.kernel(
        body,
        out_type=jax.ShapeDtypeStruct((NC, n, F), jnp.float32),
        compiler_params=pltpu.CompilerParams(needs_layout_passes=False),
        mesh=plsc.VectorSubcoreMesh(core_axis_name="c", subcore_axis_name="s"),
        scratch_types=[
            pltpu.VMEM((SB,), jnp.int32),
            pltpu.VMEM((SB,), jnp.int32),
            pltpu.VMEM((SB,), jnp.int32),
            pltpu.VMEM((SB * H,), jnp.float32),
            pltpu.VMEM((B, F), jnp.float32),
            pltpu.VMEM((B,), jnp.int32),
            pltpu.VMEM((r, F), jnp.float32),
            pltpu.VMEM_SHARED((n, F), jnp.float32),
        ],
    )


def kernel(node_feat, edge_feat, edge_index, edge_type, W_node, b_node,
           W_edge, b_edge, W_attn, b_attn, W_self):
    n = node_feat.shape[0]
    nd = node_feat.shape[1]
    r = edge_feat.shape[0]
    ed = edge_feat.shape[1]
    e = edge_type.shape[0]

    wn_flat = W_node.transpose(2, 0, 1).reshape(nd, F)
    b_flat = b_node.reshape(F)
    wa = W_attn[:, 0, :]
    eye_h = jnp.eye(H, dtype=jnp.float32)
    A = jnp.concatenate([
        (wa[:, :HD][:, :, None] * eye_h[:, None, :]).reshape(F, H),
        (wa[:, 2 * HD:][:, :, None] * eye_h[:, None, :]).reshape(F, H),
    ], axis=1)
    S = (W_self.transpose(0, 2, 1)[:, None, :, :] *
         eye_h[:, :, None, None]).transpose(0, 2, 1, 3).reshape(F, F)
    we = W_edge.transpose(1, 3, 0, 2).reshape(r, ed, F)
    be_flat = b_edge.transpose(1, 0, 2).reshape(r, F)
    a_edge_w = (wa[:, HD:2 * HD][:, :, None] * eye_h[:, None, :]).reshape(F, H)
    ba_row = b_attn[:, 0][None, :]

    BN = 2000
    grid = (n // BN,)
    node_flat, ascal, sl = pl.pallas_call(
        _proj_body,
        grid=grid,
        in_specs=[
            pl.BlockSpec((BN, nd), lambda i: (i, 0)),
            pl.BlockSpec((nd, F), lambda i: (0, 0)),
            pl.BlockSpec((F,), lambda i: (0,)),
            pl.BlockSpec((F, 2 * H), lambda i: (0, 0)),
            pl.BlockSpec((F, F), lambda i: (0, 0)),
        ],
        out_specs=[
            pl.BlockSpec((BN, F), lambda i: (i, 0)),
            pl.BlockSpec((BN, 2 * H), lambda i: (i, 0)),
            pl.BlockSpec((BN, F), lambda i: (i, 0)),
        ],
        out_shape=[
            jax.ShapeDtypeStruct((n, F), jnp.float32),
            jax.ShapeDtypeStruct((n, 2 * H), jnp.float32),
            jax.ShapeDtypeStruct((n, F), jnp.float32),
        ],
    )(node_feat, wn_flat, b_flat, A, S)

    edge_flat, aedge = pl.pallas_call(
        _edge_body,
        out_shape=[
            jax.ShapeDtypeStruct((r, F), jnp.float32),
            jax.ShapeDtypeStruct((r, H), jnp.float32),
        ],
    )(edge_feat, we, be_flat, a_edge_w, ba_row)

    src = edge_index[0]
    dst = edge_index[1]
    ascal_f = ascal.reshape(n * 8)
    aedge_f = aedge.reshape(r * H)
    zden = jnp.zeros((n * H,), jnp.float32)
    z128 = jnp.zeros((n // NS, F), jnp.float32)

    den_parts, p_edges = _make_sc_den(n, e)(src, dst, edge_type, ascal_f,
                                            aedge_f, zden)
    acc = _make_sc_msg(n, e, r)(src, dst, edge_type, p_edges, node_flat,
                                edge_flat, z128)
    den2d = den_parts.reshape(NW, n, H).transpose(1, 0, 2).reshape(n, NW * H)
    m4 = jnp.tile(eye_h, (NW, 1))

    out = pl.pallas_call(
        _final_body,
        grid=grid,
        in_specs=[
            pl.BlockSpec((NC, BN, F), lambda i: (0, i, 0)),
            pl.BlockSpec((BN, NW * H), lambda i: (i, 0)),
            pl.BlockSpec((NW * H, H), lambda i: (0, 0)),
            pl.BlockSpec((BN, F), lambda i: (i, 0)),
        ],
        out_specs=pl.BlockSpec((BN, F), lambda i: (i, 0)),
        out_shape=jax.ShapeDtypeStruct((n, F), jnp.float32),
    )(acc, den2d, m4, sl)
    return out


# speedup vs baseline: 69.3825x; 1.7816x over previous
import functools

import jax
import jax.numpy as jnp
from jax import lax
from jax.experimental import pallas as pl
from jax.experimental.pallas import tpu as pltpu
from jax.experimental.pallas import tpu_sc as plsc

H = 4
HD = 32
F = H * HD
NC = 2
NS = 16
L = 16
NW = NC * NS
SB = 2000
B = 80


def _proj_body(x_ref, wn_ref, b_ref, a_ref, s_ref, node_ref, ascal_ref, sl_ref):
    x = x_ref[...]
    node = jnp.dot(x, wn_ref[...], preferred_element_type=jnp.float32) + b_ref[...]
    node_ref[...] = node
    ascal_ref[...] = jnp.dot(node, a_ref[...], preferred_element_type=jnp.float32)
    sl_ref[...] = jnp.dot(node, s_ref[...], preferred_element_type=jnp.float32)


def _edge_body(ef_ref, we_ref, be_ref, ae_ref, ba_ref, edge_ref, aedge_ref):
    ef = ef_ref[...]
    edge = jnp.einsum("rd,rdc->rc", ef, we_ref[...],
                      preferred_element_type=jnp.float32) + be_ref[...]
    edge_ref[...] = edge
    aedge_ref[...] = (jnp.dot(edge, ae_ref[...], preferred_element_type=jnp.float32)
                      + ba_ref[...])


def _final_body(acc_ref, den_ref, m4_ref, sl_ref, out_ref):
    acc = jnp.sum(acc_ref[...], axis=0)
    den = jnp.dot(den_ref[...], m4_ref[...],
                  preferred_element_type=jnp.float32)
    inv = (1.0 / jnp.where(den > 0, den, 1.0))[:, :, None]
    v = (acc.reshape(-1, H, HD) * inv).reshape(-1, F) + sl_ref[...]
    out_ref[...] = jnp.where(v > 0, v, 0.01 * v)


def _att_p(ascal_t, aedge_t, sv, dv, tv):
    sv8 = sv * 8
    dv8 = dv * 8
    tv4 = tv * 4
    out = []
    for h in range(H):
        a = (plsc.load_gather(ascal_t, [sv8 + h])
             + plsc.load_gather(ascal_t, [dv8 + (H + h)])
             + plsc.load_gather(aedge_t, [tv4 + h]))
        a = jnp.maximum(a, a * 0.01)
        out.append(jnp.exp(a))
    return out


def _make_sc_den(n, e):
    epw = e // NW
    nsb = epw // SB

    def body(src_hbm, dst_hbm, typ_hbm, ascal_hbm, aedge_hbm, zden_hbm,
             den_out, p_out, src_f, dst_f, typ_f, p_v, ascal_t, aedge_t, den_t):
        c = lax.axis_index("c")
        s = lax.axis_index("s")
        wid = c * NS + s
        pltpu.sync_copy(ascal_hbm, ascal_t)
        pltpu.sync_copy(aedge_hbm, aedge_t)
        pltpu.sync_copy(zden_hbm, den_t)

        iota = lax.iota(jnp.int32, L)
        lane_h = jnp.bitwise_and(iota, 3)
        lane_m = iota < H

        def super_body(sb, carry):
            base = wid * epw + sb * SB
            pltpu.sync_copy(src_hbm.at[pl.ds(base, SB)], src_f)
            pltpu.sync_copy(dst_hbm.at[pl.ds(base, SB)], dst_f)
            pltpu.sync_copy(typ_hbm.at[pl.ds(base, SB)], typ_f)

            def batch_body(b, carry2):
                def att_g(g, carry3):
                    le = b * B + g * L
                    rows = g * L + iota
                    sv = plsc.load_gather(src_f, [le + iota])
                    dv = plsc.load_gather(dst_f, [le + iota])
                    tv = plsc.load_gather(typ_f, [le + iota])
                    ps = _att_p(ascal_t, aedge_t, sv, dv, tv)
                    for h in range(H):
                        plsc.store_scatter(p_v, [rows * 4 + h], ps[h])
                    return carry3
                lax.fori_loop(0, B // L, att_g, 0)

                def den_e(ei, carry3):
                    ge = jnp.full((L,), b * B + ei, jnp.int32)
                    dvs = plsc.load_gather(dst_f, [ge])
                    p4 = plsc.load_gather(p_v, [jnp.full((L,), ei * 4, jnp.int32)
                                                + lane_h])
                    plsc.addupdate_scatter(den_t, [dvs * 4 + lane_h], p4,
                                           mask=lane_m)
                    return carry3
                lax.fori_loop(0, B, den_e, 0)
                pltpu.sync_copy(
                    p_v, p_out.at[pl.ds((base + b * B) * H, B * H)])
                return carry2
            lax.fori_loop(0, SB // B, batch_body, 0)
            return carry
        lax.fori_loop(0, nsb, super_body, 0)

        pltpu.sync_copy(den_t, den_out.at[pl.ds(wid * n * H, n * H)])

    return pl.kernel(
        body,
        out_type=[
            jax.ShapeDtypeStruct((NW * n * H,), jnp.float32),
            jax.ShapeDtypeStruct((e * H,), jnp.float32),
        ],
        compiler_params=pltpu.CompilerParams(needs_layout_passes=False),
        mesh=plsc.VectorSubcoreMesh(core_axis_name="c", subcore_axis_name="s"),
        scratch_types=[
            pltpu.VMEM((SB,), jnp.int32),
            pltpu.VMEM((SB,), jnp.int32),
            pltpu.VMEM((SB,), jnp.int32),
            pltpu.VMEM((B * H,), jnp.float32),
            pltpu.VMEM((n * 8,), jnp.float32),
            pltpu.VMEM((16 * H,), jnp.float32),
            pltpu.VMEM((n * H,), jnp.float32),
        ],
    )


def _make_sc_msg(n, e, r):
    epw = e // NW
    nsb = epw // SB
    npc = n // NS

    def body(src_hbm, dst_hbm, typ_hbm, p_hbm, node_hbm,
             edge_hbm, z128_hbm, acc_out,
             src_f, dst_f, typ_f, p_f, node_v, dst_b, edge_t, acc_sh):
        c = lax.axis_index("c")
        s = lax.axis_index("s")
        wid = c * NS + s
        pltpu.sync_copy(edge_hbm, edge_t)
        pltpu.sync_copy(z128_hbm, acc_sh.at[pl.ds(s * npc, npc)])
        plsc.subcore_barrier()

        iota = lax.iota(jnp.int32, L)

        def super_body(sb, carry):
            base = wid * epw + sb * SB
            pltpu.sync_copy(src_hbm.at[pl.ds(base, SB)], src_f)
            pltpu.sync_copy(dst_hbm.at[pl.ds(base, SB)], dst_f)
            pltpu.sync_copy(typ_hbm.at[pl.ds(base, SB)], typ_f)
            pltpu.sync_copy(p_hbm.at[pl.ds(base * H, SB * H)], p_f)

            def batch_body(b, carry2):
                pltpu.sync_copy(node_hbm.at[src_f.at[pl.ds(b * B, B)]], node_v)
                for g in range(B // L):
                    dst_b[pl.ds(g * L, L)] = plsc.load_gather(
                        dst_f, [b * B + g * L + iota])

                @plsc.parallel_loop(0, B, unroll=4)
                def msg_e(ei):
                    fe = jnp.full((L,), ei, jnp.int32)
                    ge = jnp.full((L,), b * B + ei, jnp.int32)
                    tv = plsc.load_gather(typ_f, [ge])
                    ps = [plsc.load_gather(p_f, [jnp.full((L,), (b * B + ei) * H + h,
                                                          jnp.int32)])
                          for h in range(H)]
                    for j in range(F // L):
                        cols = j * L + iota
                        nv = plsc.load_gather(node_v, [fe, cols])
                        ev = plsc.load_gather(edge_t, [tv, cols])
                        plsc.store_scatter(node_v, [fe, cols],
                                           nv * ev * ps[j // (HD // L)])

                pltpu.sync_copy(node_v, acc_sh.at[dst_b], add=True)
                return carry2
            lax.fori_loop(0, SB // B, batch_body, 0)
            return carry
        lax.fori_loop(0, nsb, super_body, 0)

        plsc.subcore_barrier()

        @pl.when(s == 0)
        def _():
            pltpu.sync_copy(acc_sh, acc_out.at[c])

    return pl.kernel(
        body,
        out_type=jax.ShapeDtypeStruct((NC, n, F), jnp.float32),
        compiler_params=pltpu.CompilerParams(needs_layout_passes=False),
        mesh=plsc.VectorSubcoreMesh(core_axis_name="c", subcore_axis_name="s"),
        scratch_types=[
            pltpu.VMEM((SB,), jnp.int32),
            pltpu.VMEM((SB,), jnp.int32),
            pltpu.VMEM((SB,), jnp.int32),
            pltpu.VMEM((SB * H,), jnp.float32),
            pltpu.VMEM((B, F), jnp.float32),
            pltpu.VMEM((B,), jnp.int32),
            pltpu.VMEM((r, F), jnp.float32),
            pltpu.VMEM_SHARED((n, F), jnp.float32),
        ],
    )


def kernel(node_feat, edge_feat, edge_index, edge_type, W_node, b_node,
           W_edge, b_edge, W_attn, b_attn, W_self):
    n = node_feat.shape[0]
    nd = node_feat.shape[1]
    r = edge_feat.shape[0]
    ed = edge_feat.shape[1]
    e = edge_type.shape[0]

    wn_flat = W_node.transpose(2, 0, 1).reshape(nd, F)
    b_flat = b_node.reshape(F)
    wa = W_attn[:, 0, :]
    eye_h = jnp.eye(H, dtype=jnp.float32)
    A = jnp.concatenate([
        (wa[:, :HD][:, :, None] * eye_h[:, None, :]).reshape(F, H),
        (wa[:, 2 * HD:][:, :, None] * eye_h[:, None, :]).reshape(F, H),
    ], axis=1)
    S = (W_self.transpose(0, 2, 1)[:, None, :, :] *
         eye_h[:, :, None, None]).transpose(0, 2, 1, 3).reshape(F, F)
    we = W_edge.transpose(1, 3, 0, 2).reshape(r, ed, F)
    be_flat = b_edge.transpose(1, 0, 2).reshape(r, F)
    a_edge_w = (wa[:, HD:2 * HD][:, :, None] * eye_h[:, None, :]).reshape(F, H)
    ba_row = b_attn[:, 0][None, :]

    BN = 2000
    grid = (n // BN,)
    node_flat, ascal, sl = pl.pallas_call(
        _proj_body,
        grid=grid,
        in_specs=[
            pl.BlockSpec((BN, nd), lambda i: (i, 0)),
            pl.BlockSpec((nd, F), lambda i: (0, 0)),
            pl.BlockSpec((F,), lambda i: (0,)),
            pl.BlockSpec((F, 2 * H), lambda i: (0, 0)),
            pl.BlockSpec((F, F), lambda i: (0, 0)),
        ],
        out_specs=[
            pl.BlockSpec((BN, F), lambda i: (i, 0)),
            pl.BlockSpec((BN, 2 * H), lambda i: (i, 0)),
            pl.BlockSpec((BN, F), lambda i: (i, 0)),
        ],
        out_shape=[
            jax.ShapeDtypeStruct((n, F), jnp.float32),
            jax.ShapeDtypeStruct((n, 2 * H), jnp.float32),
            jax.ShapeDtypeStruct((n, F), jnp.float32),
        ],
    )(node_feat, wn_flat, b_flat, A, S)

    edge_flat, aedge = pl.pallas_call(
        _edge_body,
        out_shape=[
            jax.ShapeDtypeStruct((r, F), jnp.float32),
            jax.ShapeDtypeStruct((r, H), jnp.float32),
        ],
    )(edge_feat, we, be_flat, a_edge_w, ba_row)

    src = edge_index[0]
    dst = edge_index[1]
    ascal_f = ascal.reshape(n * 8)
    aedge_f = aedge.reshape(r * H)
    zden = jnp.zeros((n * H,), jnp.float32)
    z128 = jnp.zeros((n // NS, F), jnp.float32)

    den_parts, p_edges = _make_sc_den(n, e)(src, dst, edge_type, ascal_f,
                                            aedge_f, zden)
    acc = _make_sc_msg(n, e, r)(src, dst, edge_type, p_edges, node_flat,
                                edge_flat, z128)
    den2d = den_parts.reshape(NW, n, H).transpose(1, 0, 2).reshape(n, NW * H)
    m4 = jnp.tile(eye_h, (NW, 1))

    out = pl.pallas_call(
        _final_body,
        grid=grid,
        in_specs=[
            pl.BlockSpec((NC, BN, F), lambda i: (0, i, 0)),
            pl.BlockSpec((BN, NW * H), lambda i: (i, 0)),
            pl.BlockSpec((NW * H, H), lambda i: (0, 0)),
            pl.BlockSpec((BN, F), lambda i: (i, 0)),
        ],
        out_specs=pl.BlockSpec((BN, F), lambda i: (i, 0)),
        out_shape=jax.ShapeDtypeStruct((n, F), jnp.float32),
    )(acc, den2d, m4, sl)
    return out


# speedup vs baseline: 76.9508x; 1.1091x over previous
import functools

import jax
import jax.numpy as jnp
from jax import lax
from jax.experimental import pallas as pl
from jax.experimental.pallas import tpu as pltpu
from jax.experimental.pallas import tpu_sc as plsc

H = 4
HD = 32
F = H * HD
NC = 2
NS = 16
L = 16
NW = NC * NS
SB = 2000
B = 80


def _proj_body(x_ref, wn_ref, b_ref, a_ref, s_ref, node_ref, ascal_ref, sl_ref):
    x = x_ref[...]
    node = jnp.dot(x, wn_ref[...], preferred_element_type=jnp.float32) + b_ref[...]
    node_ref[...] = node
    ascal_ref[...] = jnp.dot(node, a_ref[...], preferred_element_type=jnp.float32)
    sl_ref[...] = jnp.dot(node, s_ref[...], preferred_element_type=jnp.float32)


def _edge_body(ef_ref, we_ref, be_ref, ae_ref, ba_ref, edge_ref, aedge_ref):
    ef = ef_ref[...]
    edge = jnp.einsum("rd,rdc->rc", ef, we_ref[...],
                      preferred_element_type=jnp.float32) + be_ref[...]
    edge_ref[...] = edge
    aedge_ref[...] = (jnp.dot(edge, ae_ref[...], preferred_element_type=jnp.float32)
                      + ba_ref[...])


def _final_body(acc_ref, den_ref, m4_ref, sl_ref, out_ref):
    acc = jnp.sum(acc_ref[...], axis=0)
    den = jnp.dot(den_ref[...], m4_ref[...],
                  preferred_element_type=jnp.float32)
    inv = (1.0 / jnp.where(den > 0, den, 1.0))[:, :, None]
    v = (acc.reshape(-1, H, HD) * inv).reshape(-1, F) + sl_ref[...]
    out_ref[...] = jnp.where(v > 0, v, 0.01 * v)


def _att_p(ascal_t, aedge_t, sv, dv, tv):
    sv8 = sv * 8
    dv8 = dv * 8
    tv4 = tv * 4
    out = []
    for h in range(H):
        a = (plsc.load_gather(ascal_t, [sv8 + h])
             + plsc.load_gather(ascal_t, [dv8 + (H + h)])
             + plsc.load_gather(aedge_t, [tv4 + h]))
        a = jnp.maximum(a, a * 0.01)
        out.append(jnp.exp(a))
    return out


def _make_sc_den(n, e):
    epw = e // NW
    nsb = epw // SB

    def body(src_hbm, dst_hbm, typ_hbm, ascal_hbm, aedge_hbm, zden_hbm,
             den_out, p_out, src_f, dst_f, typ_f, p_v, ascal_t, aedge_t, den_t):
        c = lax.axis_index("c")
        s = lax.axis_index("s")
        wid = c * NS + s
        pltpu.sync_copy(ascal_hbm, ascal_t)
        pltpu.sync_copy(aedge_hbm, aedge_t)
        pltpu.sync_copy(zden_hbm, den_t)

        iota = lax.iota(jnp.int32, L)
        lane_h = jnp.bitwise_and(iota, 3)
        lane_m = iota < H

        def super_body(sb, carry):
            base = wid * epw + sb * SB
            pltpu.sync_copy(src_hbm.at[pl.ds(base, SB)], src_f)
            pltpu.sync_copy(dst_hbm.at[pl.ds(base, SB)], dst_f)
            pltpu.sync_copy(typ_hbm.at[pl.ds(base, SB)], typ_f)

            def batch_body(b, carry2):
                @plsc.parallel_loop(0, B // L, unroll=2)
                def att_g(g):
                    le = b * B + g * L
                    rows = g * L + iota
                    sv = plsc.load_gather(src_f, [le + iota])
                    dv = plsc.load_gather(dst_f, [le + iota])
                    tv = plsc.load_gather(typ_f, [le + iota])
                    ps = _att_p(ascal_t, aedge_t, sv, dv, tv)
                    for h in range(H):
                        plsc.store_scatter(p_v, [rows * 4 + h], ps[h])

                @plsc.parallel_loop(0, B, unroll=4)
                def den_e(ei):
                    ge = jnp.full((L,), ei, jnp.int32)
                    dvs = plsc.load_gather(dst_f, [ge + b * B])
                    p4 = plsc.load_gather(p_v, [jnp.full((L,), ei * 4, jnp.int32)
                                                + lane_h])
                    plsc.addupdate_scatter(den_t, [dvs * 4 + lane_h], p4,
                                           mask=lane_m)
                pltpu.sync_copy(
                    p_v, p_out.at[pl.ds((base + b * B) * H, B * H)])
                return carry2
            lax.fori_loop(0, SB // B, batch_body, 0)
            return carry
        lax.fori_loop(0, nsb, super_body, 0)

        pltpu.sync_copy(den_t, den_out.at[pl.ds(wid * n * H, n * H)])

    return pl.kernel(
        body,
        out_type=[
            jax.ShapeDtypeStruct((NW * n * H,), jnp.float32),
            jax.ShapeDtypeStruct((e * H,), jnp.float32),
        ],
        compiler_params=pltpu.CompilerParams(needs_layout_passes=False),
        mesh=plsc.VectorSubcoreMesh(core_axis_name="c", subcore_axis_name="s"),
        scratch_types=[
            pltpu.VMEM((SB,), jnp.int32),
            pltpu.VMEM((SB,), jnp.int32),
            pltpu.VMEM((SB,), jnp.int32),
            pltpu.VMEM((B * H,), jnp.float32),
            pltpu.VMEM((n * 8,), jnp.float32),
            pltpu.VMEM((16 * H,), jnp.float32),
            pltpu.VMEM((n * H,), jnp.float32),
        ],
    )


def _make_sc_msg(n, e, r):
    epw = e // NW
    nsb = epw // SB
    npc = n // NS

    def body(src_hbm, dst_hbm, typ_hbm, p_hbm, node_hbm,
             edge_hbm, z128_hbm, acc_out,
             src_f, dst_f, typ_f, p_f, node_v, dst_b, edge_t, acc_sh):
        c = lax.axis_index("c")
        s = lax.axis_index("s")
        wid = c * NS + s
        pltpu.sync_copy(edge_hbm, edge_t)
        pltpu.sync_copy(z128_hbm, acc_sh.at[pl.ds(s * npc, npc)])
        plsc.subcore_barrier()

        iota = lax.iota(jnp.int32, L)

        def super_body(sb, carry):
            base = wid * epw + sb * SB
            pltpu.sync_copy(src_hbm.at[pl.ds(base, SB)], src_f)
            pltpu.sync_copy(dst_hbm.at[pl.ds(base, SB)], dst_f)
            pltpu.sync_copy(typ_hbm.at[pl.ds(base, SB)], typ_f)
            pltpu.sync_copy(p_hbm.at[pl.ds(base * H, SB * H)], p_f)

            def batch_body(b, carry2):
                pltpu.sync_copy(node_hbm.at[src_f.at[pl.ds(b * B, B)]], node_v)
                for g in range(B // L):
                    dst_b[pl.ds(g * L, L)] = plsc.load_gather(
                        dst_f, [b * B + g * L + iota])

                @plsc.parallel_loop(0, B, unroll=4)
                def msg_e(ei):
                    fe = jnp.full((L,), ei, jnp.int32)
                    ge = jnp.full((L,), b * B + ei, jnp.int32)
                    tv = plsc.load_gather(typ_f, [ge])
                    ps = [plsc.load_gather(p_f, [jnp.full((L,), (b * B + ei) * H + h,
                                                          jnp.int32)])
                          for h in range(H)]
                    for j in range(F // L):
                        cols = j * L + iota
                        nv = plsc.load_gather(node_v, [fe, cols])
                        ev = plsc.load_gather(edge_t, [tv, cols])
                        plsc.store_scatter(node_v, [fe, cols],
                                           nv * ev * ps[j // (HD // L)])

                pltpu.sync_copy(node_v, acc_sh.at[dst_b], add=True)
                return carry2
            lax.fori_loop(0, SB // B, batch_body, 0)
            return carry
        lax.fori_loop(0, nsb, super_body, 0)

        plsc.subcore_barrier()

        @pl.when(s == 0)
        def _():
            pltpu.sync_copy(acc_sh, acc_out.at[c])

    return pl.kernel(
        body,
        out_type=jax.ShapeDtypeStruct((NC, n, F), jnp.float32),
        compiler_params=pltpu.CompilerParams(needs_layout_passes=False),
        mesh=plsc.VectorSubcoreMesh(core_axis_name="c", subcore_axis_name="s"),
        scratch_types=[
            pltpu.VMEM((SB,), jnp.int32),
            pltpu.VMEM((SB,), jnp.int32),
            pltpu.VMEM((SB,), jnp.int32),
            pltpu.VMEM((SB * H,), jnp.float32),
            pltpu.VMEM((B, F), jnp.float32),
            pltpu.VMEM((B,), jnp.int32),
            pltpu.VMEM((r, F), jnp.float32),
            pltpu.VMEM_SHARED((n, F), jnp.float32),
        ],
    )


def kernel(node_feat, edge_feat, edge_index, edge_type, W_node, b_node,
           W_edge, b_edge, W_attn, b_attn, W_self):
    n = node_feat.shape[0]
    nd = node_feat.shape[1]
    r = edge_feat.shape[0]
    ed = edge_feat.shape[1]
    e = edge_type.shape[0]

    wn_flat = W_node.transpose(2, 0, 1).reshape(nd, F)
    b_flat = b_node.reshape(F)
    wa = W_attn[:, 0, :]
    eye_h = jnp.eye(H, dtype=jnp.float32)
    A = jnp.concatenate([
        (wa[:, :HD][:, :, None] * eye_h[:, None, :]).reshape(F, H),
        (wa[:, 2 * HD:][:, :, None] * eye_h[:, None, :]).reshape(F, H),
    ], axis=1)
    S = (W_self.transpose(0, 2, 1)[:, None, :, :] *
         eye_h[:, :, None, None]).transpose(0, 2, 1, 3).reshape(F, F)
    we = W_edge.transpose(1, 3, 0, 2).reshape(r, ed, F)
    be_flat = b_edge.transpose(1, 0, 2).reshape(r, F)
    a_edge_w = (wa[:, HD:2 * HD][:, :, None] * eye_h[:, None, :]).reshape(F, H)
    ba_row = b_attn[:, 0][None, :]

    BN = 2000
    grid = (n // BN,)
    node_flat, ascal, sl = pl.pallas_call(
        _proj_body,
        grid=grid,
        in_specs=[
            pl.BlockSpec((BN, nd), lambda i: (i, 0)),
            pl.BlockSpec((nd, F), lambda i: (0, 0)),
            pl.BlockSpec((F,), lambda i: (0,)),
            pl.BlockSpec((F, 2 * H), lambda i: (0, 0)),
            pl.BlockSpec((F, F), lambda i: (0, 0)),
        ],
        out_specs=[
            pl.BlockSpec((BN, F), lambda i: (i, 0)),
            pl.BlockSpec((BN, 2 * H), lambda i: (i, 0)),
            pl.BlockSpec((BN, F), lambda i: (i, 0)),
        ],
        out_shape=[
            jax.ShapeDtypeStruct((n, F), jnp.float32),
            jax.ShapeDtypeStruct((n, 2 * H), jnp.float32),
            jax.ShapeDtypeStruct((n, F), jnp.float32),
        ],
    )(node_feat, wn_flat, b_flat, A, S)

    edge_flat, aedge = pl.pallas_call(
        _edge_body,
        out_shape=[
            jax.ShapeDtypeStruct((r, F), jnp.float32),
            jax.ShapeDtypeStruct((r, H), jnp.float32),
        ],
    )(edge_feat, we, be_flat, a_edge_w, ba_row)

    src = edge_index[0]
    dst = edge_index[1]
    ascal_f = ascal.reshape(n * 8)
    aedge_f = aedge.reshape(r * H)
    zden = jnp.zeros((n * H,), jnp.float32)
    z128 = jnp.zeros((n // NS, F), jnp.float32)

    den_parts, p_edges = _make_sc_den(n, e)(src, dst, edge_type, ascal_f,
                                            aedge_f, zden)
    acc = _make_sc_msg(n, e, r)(src, dst, edge_type, p_edges, node_flat,
                                edge_flat, z128)
    den2d = den_parts.reshape(NW, n, H).transpose(1, 0, 2).reshape(n, NW * H)
    m4 = jnp.tile(eye_h, (NW, 1))

    out = pl.pallas_call(
        _final_body,
        grid=grid,
        in_specs=[
            pl.BlockSpec((NC, BN, F), lambda i: (0, i, 0)),
            pl.BlockSpec((BN, NW * H), lambda i: (i, 0)),
            pl.BlockSpec((NW * H, H), lambda i: (0, 0)),
            pl.BlockSpec((BN, F), lambda i: (i, 0)),
        ],
        out_specs=pl.BlockSpec((BN, F), lambda i: (i, 0)),
        out_shape=jax.ShapeDtypeStruct((n, F), jnp.float32),
    )(acc, den2d, m4, sl)
    return out


# speedup vs baseline: 83.4168x; 1.0840x over previous
import functools

import jax
import jax.numpy as jnp
from jax import lax
from jax.experimental import pallas as pl
from jax.experimental.pallas import tpu as pltpu
from jax.experimental.pallas import tpu_sc as plsc

H = 4
HD = 32
F = H * HD
NC = 2
NS = 16
L = 16
NW = NC * NS
SB = 2000
B = 80


def _proj_body(x_ref, wn_ref, b_ref, a_ref, s_ref, node_ref, ascal_ref, sl_ref):
    x = x_ref[...]
    node = jnp.dot(x, wn_ref[...], preferred_element_type=jnp.float32) + b_ref[...]
    node_ref[...] = node
    ascal_ref[...] = jnp.dot(node, a_ref[...], preferred_element_type=jnp.float32)
    sl_ref[...] = jnp.dot(node, s_ref[...], preferred_element_type=jnp.float32)


def _edge_body(ef_ref, we_ref, be_ref, ae_ref, ba_ref, edge_ref, aedge_ref):
    ef = ef_ref[...]
    edge = jnp.einsum("rd,rdc->rc", ef, we_ref[...],
                      preferred_element_type=jnp.float32) + be_ref[...]
    edge_ref[...] = edge
    aedge_ref[...] = (jnp.dot(edge, ae_ref[...], preferred_element_type=jnp.float32)
                      + ba_ref[...])


def _final_body(acc_ref, den_ref, m4_ref, sl_ref, out_ref):
    acc = jnp.sum(acc_ref[...], axis=0)
    den = jnp.dot(den_ref[...], m4_ref[...],
                  preferred_element_type=jnp.float32)
    inv = (1.0 / jnp.where(den > 0, den, 1.0))[:, :, None]
    v = (acc.reshape(-1, H, HD) * inv).reshape(-1, F) + sl_ref[...]
    out_ref[...] = jnp.where(v > 0, v, 0.01 * v)


def _att_p(ascal_t, aedge_t, sv, dv, tv):
    sv8 = sv * 8
    dv8 = dv * 8
    tv4 = tv * 4
    out = []
    for h in range(H):
        a = (plsc.load_gather(ascal_t, [sv8 + h])
             + plsc.load_gather(ascal_t, [dv8 + (H + h)])
             + plsc.load_gather(aedge_t, [tv4 + h]))
        a = jnp.maximum(a, a * 0.01)
        out.append(jnp.exp(a))
    return out


def _make_sc_den(n, e):
    epw = e // NW
    nsb = epw // SB

    def body(src_hbm, dst_hbm, typ_hbm, ascal_hbm, aedge_hbm, zden_hbm,
             den_out, p_out, src_f, dst_f, typ_f, p_v, ascal_t, aedge_t, den_t):
        c = lax.axis_index("c")
        s = lax.axis_index("s")
        wid = c * NS + s
        pltpu.sync_copy(ascal_hbm, ascal_t)
        pltpu.sync_copy(aedge_hbm, aedge_t)
        pltpu.sync_copy(zden_hbm, den_t)

        iota = lax.iota(jnp.int32, L)
        lane_h = jnp.bitwise_and(iota, 3)
        lane_m = iota < H

        def super_body(sb, carry):
            base = wid * epw + sb * SB
            pltpu.sync_copy(src_hbm.at[pl.ds(base, SB)], src_f)
            pltpu.sync_copy(dst_hbm.at[pl.ds(base, SB)], dst_f)
            pltpu.sync_copy(typ_hbm.at[pl.ds(base, SB)], typ_f)

            def batch_body(b, carry2):
                @plsc.parallel_loop(0, B // L, unroll=2)
                def att_g(g):
                    le = b * B + g * L
                    rows = g * L + iota
                    sv = plsc.load_gather(src_f, [le + iota])
                    dv = plsc.load_gather(dst_f, [le + iota])
                    tv = plsc.load_gather(typ_f, [le + iota])
                    ps = _att_p(ascal_t, aedge_t, sv, dv, tv)
                    for h in range(H):
                        plsc.store_scatter(p_v, [rows * 4 + h], ps[h])

                @plsc.parallel_loop(0, B, unroll=4)
                def den_e(ei):
                    ge = jnp.full((L,), ei, jnp.int32)
                    dvs = plsc.load_gather(dst_f, [ge + b * B])
                    p4 = plsc.load_gather(p_v, [jnp.full((L,), ei * 4, jnp.int32)
                                                + lane_h])
                    plsc.addupdate_scatter(den_t, [dvs * 4 + lane_h], p4,
                                           mask=lane_m)
                pltpu.sync_copy(
                    p_v, p_out.at[pl.ds((base + b * B) * H, B * H)])
                return carry2
            lax.fori_loop(0, SB // B, batch_body, 0)
            return carry
        lax.fori_loop(0, nsb, super_body, 0)

        pltpu.sync_copy(den_t, den_out.at[pl.ds(wid * n * H, n * H)])

    return pl.kernel(
        body,
        out_type=[
            jax.ShapeDtypeStruct((NW * n * H,), jnp.float32),
            jax.ShapeDtypeStruct((e * H,), jnp.float32),
        ],
        compiler_params=pltpu.CompilerParams(needs_layout_passes=False),
        mesh=plsc.VectorSubcoreMesh(core_axis_name="c", subcore_axis_name="s"),
        scratch_types=[
            pltpu.VMEM((SB,), jnp.int32),
            pltpu.VMEM((SB,), jnp.int32),
            pltpu.VMEM((SB,), jnp.int32),
            pltpu.VMEM((B * H,), jnp.float32),
            pltpu.VMEM((n * 8,), jnp.float32),
            pltpu.VMEM((16 * H,), jnp.float32),
            pltpu.VMEM((n * H,), jnp.float32),
        ],
    )


def _make_sc_msg(n, e, r):
    epw = e // NW
    nsb = epw // SB
    npc = n // NS

    def body(src_hbm, dst_hbm, typ_hbm, p_hbm, node_hbm,
             edge_hbm, z128_hbm, acc_out,
             src_f, dst_f, typ_f, p_f, node_v, dst_ba, dst_bb, edge_t,
             acc_sh, gsa, gsb, ssa, ssb):
        c = lax.axis_index("c")
        s = lax.axis_index("s")
        wid = c * NS + s
        pltpu.sync_copy(edge_hbm, edge_t)
        pltpu.sync_copy(z128_hbm, acc_sh.at[pl.ds(s * npc, npc)])
        plsc.subcore_barrier()

        iota = lax.iota(jnp.int32, L)

        def super_body(sb, carry):
            base = wid * epw + sb * SB
            pltpu.sync_copy(src_hbm.at[pl.ds(base, SB)], src_f)
            pltpu.sync_copy(dst_hbm.at[pl.ds(base, SB)], dst_f)
            pltpu.sync_copy(typ_hbm.at[pl.ds(base, SB)], typ_f)
            pltpu.sync_copy(p_hbm.at[pl.ds(base * H, SB * H)], p_f)

            def batch_body(b, carry2):
                HA = 48
                ga = pltpu.async_copy(
                    node_hbm.at[src_f.at[pl.ds(b * B, HA)]],
                    node_v.at[pl.ds(0, HA)], gsa)
                gb = pltpu.async_copy(
                    node_hbm.at[src_f.at[pl.ds(b * B + HA, B - HA)]],
                    node_v.at[pl.ds(HA, B - HA)], gsb)
                for g in range(HA // L):
                    dst_ba[pl.ds(g * L, L)] = plsc.load_gather(
                        dst_f, [b * B + g * L + iota])
                for g in range(HA // L, B // L):
                    dst_bb[pl.ds(g * L - HA, L)] = plsc.load_gather(
                        dst_f, [b * B + g * L + iota])

                def msg_e(ei):
                    fe = jnp.full((L,), ei, jnp.int32)
                    ge = jnp.full((L,), b * B + ei, jnp.int32)
                    tv = plsc.load_gather(typ_f, [ge])
                    ps = [plsc.load_gather(p_f, [jnp.full((L,), (b * B + ei) * H + h,
                                                          jnp.int32)])
                          for h in range(H)]
                    for j in range(F // L):
                        cols = j * L + iota
                        nv = plsc.load_gather(node_v, [fe, cols])
                        ev = plsc.load_gather(edge_t, [tv, cols])
                        plsc.store_scatter(node_v, [fe, cols],
                                           nv * ev * ps[j // (HD // L)])

                ga.wait()
                plsc.parallel_loop(0, HA, unroll=4)(msg_e)
                sa = pltpu.async_copy(node_v.at[pl.ds(0, HA)],
                                      acc_sh.at[dst_ba], ssa, add=True)
                gb.wait()
                plsc.parallel_loop(HA, B, unroll=4)(msg_e)
                sb = pltpu.async_copy(node_v.at[pl.ds(HA, B - HA)],
                                      acc_sh.at[dst_bb], ssb, add=True)
                sa.wait()
                sb.wait()
                return carry2
            lax.fori_loop(0, SB // B, batch_body, 0)
            return carry
        lax.fori_loop(0, nsb, super_body, 0)

        plsc.subcore_barrier()

        @pl.when(s == 0)
        def _():
            pltpu.sync_copy(acc_sh, acc_out.at[c])

    return pl.kernel(
        body,
        out_type=jax.ShapeDtypeStruct((NC, n, F), jnp.float32),
        compiler_params=pltpu.CompilerParams(needs_layout_passes=False),
        mesh=plsc.VectorSubcoreMesh(core_axis_name="c", subcore_axis_name="s"),
        scratch_types=[
            pltpu.VMEM((SB,), jnp.int32),
            pltpu.VMEM((SB,), jnp.int32),
            pltpu.VMEM((SB,), jnp.int32),
            pltpu.VMEM((SB * H,), jnp.float32),
            pltpu.VMEM((B, F), jnp.float32),
            pltpu.VMEM((48,), jnp.int32),
            pltpu.VMEM((B - 48,), jnp.int32),
            pltpu.VMEM((r, F), jnp.float32),
            pltpu.VMEM_SHARED((n, F), jnp.float32),
            pltpu.SemaphoreType.DMA,
            pltpu.SemaphoreType.DMA,
            pltpu.SemaphoreType.DMA,
            pltpu.SemaphoreType.DMA,
        ],
    )


def kernel(node_feat, edge_feat, edge_index, edge_type, W_node, b_node,
           W_edge, b_edge, W_attn, b_attn, W_self):
    n = node_feat.shape[0]
    nd = node_feat.shape[1]
    r = edge_feat.shape[0]
    ed = edge_feat.shape[1]
    e = edge_type.shape[0]

    wn_flat = W_node.transpose(2, 0, 1).reshape(nd, F)
    b_flat = b_node.reshape(F)
    wa = W_attn[:, 0, :]
    eye_h = jnp.eye(H, dtype=jnp.float32)
    A = jnp.concatenate([
        (wa[:, :HD][:, :, None] * eye_h[:, None, :]).reshape(F, H),
        (wa[:, 2 * HD:][:, :, None] * eye_h[:, None, :]).reshape(F, H),
    ], axis=1)
    S = (W_self.transpose(0, 2, 1)[:, None, :, :] *
         eye_h[:, :, None, None]).transpose(0, 2, 1, 3).reshape(F, F)
    we = W_edge.transpose(1, 3, 0, 2).reshape(r, ed, F)
    be_flat = b_edge.transpose(1, 0, 2).reshape(r, F)
    a_edge_w = (wa[:, HD:2 * HD][:, :, None] * eye_h[:, None, :]).reshape(F, H)
    ba_row = b_attn[:, 0][None, :]

    BN = 2000
    grid = (n // BN,)
    node_flat, ascal, sl = pl.pallas_call(
        _proj_body,
        grid=grid,
        in_specs=[
            pl.BlockSpec((BN, nd), lambda i: (i, 0)),
            pl.BlockSpec((nd, F), lambda i: (0, 0)),
            pl.BlockSpec((F,), lambda i: (0,)),
            pl.BlockSpec((F, 2 * H), lambda i: (0, 0)),
            pl.BlockSpec((F, F), lambda i: (0, 0)),
        ],
        out_specs=[
            pl.BlockSpec((BN, F), lambda i: (i, 0)),
            pl.BlockSpec((BN, 2 * H), lambda i: (i, 0)),
            pl.BlockSpec((BN, F), lambda i: (i, 0)),
        ],
        out_shape=[
            jax.ShapeDtypeStruct((n, F), jnp.float32),
            jax.ShapeDtypeStruct((n, 2 * H), jnp.float32),
            jax.ShapeDtypeStruct((n, F), jnp.float32),
        ],
    )(node_feat, wn_flat, b_flat, A, S)

    edge_flat, aedge = pl.pallas_call(
        _edge_body,
        out_shape=[
            jax.ShapeDtypeStruct((r, F), jnp.float32),
            jax.ShapeDtypeStruct((r, H), jnp.float32),
        ],
    )(edge_feat, we, be_flat, a_edge_w, ba_row)

    src = edge_index[0]
    dst = edge_index[1]
    ascal_f = ascal.reshape(n * 8)
    aedge_f = aedge.reshape(r * H)
    zden = jnp.zeros((n * H,), jnp.float32)
    z128 = jnp.zeros((n // NS, F), jnp.float32)

    den_parts, p_edges = _make_sc_den(n, e)(src, dst, edge_type, ascal_f,
                                            aedge_f, zden)
    acc = _make_sc_msg(n, e, r)(src, dst, edge_type, p_edges, node_flat,
                                edge_flat, z128)
    den2d = den_parts.reshape(NW, n, H).transpose(1, 0, 2).reshape(n, NW * H)
    m4 = jnp.tile(eye_h, (NW, 1))

    out = pl.pallas_call(
        _final_body,
        grid=grid,
        in_specs=[
            pl.BlockSpec((NC, BN, F), lambda i: (0, i, 0)),
            pl.BlockSpec((BN, NW * H), lambda i: (i, 0)),
            pl.BlockSpec((NW * H, H), lambda i: (0, 0)),
            pl.BlockSpec((BN, F), lambda i: (i, 0)),
        ],
        out_specs=pl.BlockSpec((BN, F), lambda i: (i, 0)),
        out_shape=jax.ShapeDtypeStruct((n, F), jnp.float32),
    )(acc, den2d, m4, sl)
    return out
